# baseline (speedup 1.0000x reference)
import functools

import jax
import jax.numpy as jnp
from jax import lax
from jax.experimental import pallas as pl
from jax.experimental.pallas import tpu as pltpu
from jax.experimental.pallas import tpu_sc as plsc

_B, _C, _H, _W = 8, 4, 512, 512
_CE, _HE, _WE = 128, 128, 128
_NSEG = 64
_NPIX = _HE * _WE
_KD = 4
_HEB = _HE // _KD
_HBB = _H // _KD
_NW = 32
_PPW = _NPIX * _B // _NW
_RPW = _PPW // 16


def _d_body(o_ref, i_ref, m_ref, e_ref, d_ref,
            err_ref, sums_ref, acc_ref):
    b = pl.program_id(0)
    k = pl.program_id(1)

    @pl.when((b == 0) & (k == 0))
    def _init():
        acc_ref[0] = 0.0
        acc_ref[1] = 0.0

    m = m_ref[0, 0]
    o = o_ref[0]
    x = i_ref[0]
    t = jnp.where(m[None] >= 0.5, 0.0, x)
    dd = o - t
    mse = jnp.sum(dd * dd, axis=0)
    w = (m > 0.0).astype(jnp.float32)
    acc_ref[0] += jnp.sum(mse * w)
    acc_ref[1] += jnp.sum(w)

    de = e_ref[0] - d_ref[0]
    err_ref[0] = jnp.sum(de * de, axis=0) * (1.0 / _CE)

    @pl.when((b == _B - 1) & (k == _KD - 1))
    def _fini():
        sums_ref[0] = acc_ref[0]
        sums_ref[1] = acc_ref[1]


def _dense_pass(outputs, inputs, masks, enc1, dec1):
    return pl.pallas_call(
        _d_body,
        grid=(_B, _KD),
        in_specs=[
            pl.BlockSpec((1, _C, _HBB, _W), lambda b, k: (b, 0, k, 0)),
            pl.BlockSpec((1, _C, _HBB, _W), lambda b, k: (b, 0, k, 0)),
            pl.BlockSpec((1, 1, _HBB, _W), lambda b, k: (b, 0, k, 0)),
            pl.BlockSpec((1, _CE, _HEB, _WE), lambda b, k: (b, 0, k, 0)),
            pl.BlockSpec((1, _CE, _HEB, _WE), lambda b, k: (b, 0, k, 0)),
        ],
        out_specs=[
            pl.BlockSpec((1, _HEB, _WE), lambda b, k: (b, k, 0)),
            pl.BlockSpec(memory_space=pltpu.SMEM),
        ],
        out_shape=[
            jax.ShapeDtypeStruct((_B, _HE, _WE), jnp.float32),
            jax.ShapeDtypeStruct((2,), jnp.float32),
        ],
        scratch_shapes=[pltpu.SMEM((2,), jnp.float32)],
    )(outputs, inputs, masks, enc1, dec1)


_RROWS = _PPW // _WE


def _sc_body(seg_hbm, mask_hbm, err_hbm, out_hbm, seg_v, mask_v, err_v, table,
             sem):
    c = lax.axis_index("c")
    s = lax.axis_index("s")
    wid = s * 2 + c
    b = wid // 4
    q = wid % 4
    cp_s = pltpu.async_copy(
        seg_hbm.at[b, pl.ds(q * _RROWS, _RROWS), pl.ds(0, _W)], seg_v, sem
    )
    cp_m = pltpu.async_copy(
        mask_hbm.at[b, pl.ds(q * _RROWS, _RROWS), pl.ds(0, _W)], mask_v, sem
    )
    cp_e = pltpu.async_copy(err_hbm.at[pl.ds(wid * _RROWS, _RROWS)], err_v, sem)

    zf = jnp.zeros((16,), jnp.float32)
    for r in range(3 * _NSEG):
        table[pl.ds(r * 16, 16)] = zf

    cp_s.wait()
    cp_m.wait()
    cp_e.wait()

    lane = lax.iota(jnp.int32, 16)
    ones_f = jnp.full((16,), 1.0, jnp.float32)

    def step(r, l):
        row = jnp.full((16,), r, jnp.int32)
        col4 = lane * 4 + (l * 64)
        sg = plsc.load_gather(seg_v, [row, col4]).astype(jnp.int32)
        m = plsc.load_gather(mask_v, [row, col4])
        e = err_v[r, pl.ds(l * 16, 16)]
        pos = jnp.where((m > 0.0) & (m < 0.5), 1.0, 0.0)
        base = sg * 16 + lane
        plsc.addupdate_scatter(table, [base], ones_f)
        plsc.addupdate_scatter(table, [base + (_NSEG * 16)], e)
        plsc.addupdate_scatter(table, [base + (2 * _NSEG * 16)], pos)

    @plsc.parallel_loop(0, _RROWS, unroll=2)
    def _loop(r):
        for l in range(_WE // 16):
            step(r, l)

    pltpu.sync_copy(table, out_hbm.at[wid])


def _sc_segsum(seg3, mask3, err2d):
    mesh = plsc.VectorSubcoreMesh(core_axis_name="c", subcore_axis_name="s")
    fn = functools.partial(
        pl.kernel,
        mesh=mesh,
        compiler_params=pltpu.CompilerParams(needs_layout_passes=False),
        out_type=jax.ShapeDtypeStruct((_NW, 3 * _NSEG * 16), jnp.float32),
        scratch_types=[
            pltpu.VMEM((_RROWS, _W), jnp.float32),
            pltpu.VMEM((_RROWS, _W), jnp.float32),
            pltpu.VMEM((_RROWS, _WE), jnp.float32),
            pltpu.VMEM((3 * _NSEG * 16,), jnp.float32),
            pltpu.SemaphoreType.DMA,
        ],
    )(_sc_body)
    return fn(seg3, mask3, err2d)


def _epi_body(p_ref, s_ref, o_ref):
    p = p_ref[...]
    r = lax.broadcasted_iota(jnp.int32, (3 * _NSEG * 16, 3 * _NSEG), 0)
    c = lax.broadcasted_iota(jnp.int32, (3 * _NSEG * 16, 3 * _NSEG), 1)
    gm = (r // 16 == c).astype(jnp.float32)
    t = jax.lax.dot(p, gm, precision=jax.lax.Precision.HIGHEST)
    num = 0.0
    den = 0.0
    for b in range(_B):
        g = t[4 * b] + t[4 * b + 1] + t[4 * b + 2] + t[4 * b + 3]
        counts = g[0:_NSEG]
        errs = g[_NSEG:2 * _NSEG]
        pos = g[2 * _NSEG:3 * _NSEG]
        cm = jnp.maximum(counts, 1.0)
        mean_err = errs / cm
        valid = (counts / _NPIX) >= 0.01
        is_pos = (pos / cm) > 0.01
        sel = jnp.where(valid & is_pos, 1.0, 0.0)
        num += jnp.sum(mean_err * sel)
        den += jnp.sum(sel)
    o_ref[0] = s_ref[0] / jnp.maximum(s_ref[1], 1.0) + num / jnp.maximum(den, 1.0)


def _epilogue(partials, sums):
    return pl.pallas_call(
        _epi_body,
        in_specs=[
            pl.BlockSpec(memory_space=pltpu.VMEM),
            pl.BlockSpec(memory_space=pltpu.SMEM),
        ],
        out_specs=pl.BlockSpec(memory_space=pltpu.SMEM),
        out_shape=jax.ShapeDtypeStruct((1,), jnp.float32),
    )(partials, sums)


def kernel(outputs, inputs, enc1, dec1, masks, segs, confidence, iteration, epoch):
    err, sums = _dense_pass(outputs, inputs, masks, enc1, dec1)
    seg3 = segs.reshape(_B, _HE, 4 * _W)
    mask3 = masks.reshape(_B, _HE, 4 * _W)
    partials = _sc_segsum(seg3, mask3, err.reshape(_B * _NPIX // _WE, _WE))
    loss = _epilogue(partials, sums)
    return loss[0]


# speedup vs baseline: 1.1987x; 1.1987x over previous
import functools

import jax
import jax.numpy as jnp
from jax import lax
from jax.experimental import pallas as pl
from jax.experimental.pallas import tpu as pltpu
from jax.experimental.pallas import tpu_sc as plsc

_B, _C, _H, _W = 8, 4, 512, 512
_CE, _HE, _WE = 128, 128, 128
_NSEG = 64
_NPIX = _HE * _WE
_KD = 4
_HEB = _HE // _KD
_HBB = _H // _KD
_NW = 32
_PPW = _NPIX * _B // _NW
_RPW = _PPW // 16


def _d_body(o_ref, i_ref, m_ref, e_ref, d_ref, s_ref,
            err_ref, sgr_ref, mr_ref, sums_ref, acc_ref):
    b = pl.program_id(0)
    k = pl.program_id(1)

    @pl.when((b == 0) & (k == 0))
    def _init():
        acc_ref[0] = 0.0
        acc_ref[1] = 0.0

    m = m_ref[0, 0]
    o = o_ref[0]
    x = i_ref[0]
    t = jnp.where(m[None] >= 0.5, 0.0, x)
    dd = o - t
    mse = jnp.sum(dd * dd, axis=0)
    w = (m > 0.0).astype(jnp.float32)
    acc_ref[0] += jnp.sum(mse * w)
    acc_ref[1] += jnp.sum(w)

    de = e_ref[0] - d_ref[0]
    err_ref[0] = jnp.sum(de * de, axis=0) * (1.0 / _CE)
    sgr_ref[0] = s_ref[0, 0].reshape(_HEB, 4, _W)[:, 0]
    mr_ref[0] = m.reshape(_HEB, 4, _W)[:, 0]

    @pl.when((b == _B - 1) & (k == _KD - 1))
    def _fini():
        sums_ref[0] = acc_ref[0]
        sums_ref[1] = acc_ref[1]


def _dense_pass(outputs, inputs, masks, enc1, dec1, segs):
    return pl.pallas_call(
        _d_body,
        grid=(_B, _KD),
        in_specs=[
            pl.BlockSpec((1, _C, _HBB, _W), lambda b, k: (b, 0, k, 0)),
            pl.BlockSpec((1, _C, _HBB, _W), lambda b, k: (b, 0, k, 0)),
            pl.BlockSpec((1, 1, _HBB, _W), lambda b, k: (b, 0, k, 0)),
            pl.BlockSpec((1, _CE, _HEB, _WE), lambda b, k: (b, 0, k, 0)),
            pl.BlockSpec((1, _CE, _HEB, _WE), lambda b, k: (b, 0, k, 0)),
            pl.BlockSpec((1, 1, _HBB, _W), lambda b, k: (b, 0, k, 0)),
        ],
        out_specs=[
            pl.BlockSpec((1, _HEB, _WE), lambda b, k: (b, k, 0)),
            pl.BlockSpec((1, _HEB, _W), lambda b, k: (b, k, 0)),
            pl.BlockSpec((1, _HEB, _W), lambda b, k: (b, k, 0)),
            pl.BlockSpec(memory_space=pltpu.SMEM),
        ],
        out_shape=[
            jax.ShapeDtypeStruct((_B, _HE, _WE), jnp.float32),
            jax.ShapeDtypeStruct((_B, _HE, _W), jnp.float32),
            jax.ShapeDtypeStruct((_B, _HE, _W), jnp.float32),
            jax.ShapeDtypeStruct((2,), jnp.float32),
        ],
        scratch_shapes=[pltpu.SMEM((2,), jnp.float32)],
    )(outputs, inputs, masks, enc1, dec1, segs)


_RROWS = _PPW // _WE


def _sc_body(seg_hbm, mask_hbm, err_hbm, out_hbm, seg_v, mask_v, err_v, table,
             sem):
    c = lax.axis_index("c")
    s = lax.axis_index("s")
    wid = s * 2 + c
    row0 = wid * _RROWS
    cp_s = pltpu.async_copy(seg_hbm.at[pl.ds(row0, _RROWS)], seg_v, sem)
    cp_m = pltpu.async_copy(mask_hbm.at[pl.ds(row0, _RROWS)], mask_v, sem)
    cp_e = pltpu.async_copy(err_hbm.at[pl.ds(row0, _RROWS)], err_v, sem)

    zf = jnp.zeros((16,), jnp.float32)
    for r in range(3 * _NSEG):
        table[pl.ds(r * 16, 16)] = zf

    cp_s.wait()
    cp_m.wait()
    cp_e.wait()

    lane = lax.iota(jnp.int32, 16)
    ones_f = jnp.full((16,), 1.0, jnp.float32)

    def step(r, l):
        row = jnp.full((16,), r, jnp.int32)
        col4 = lane * 4 + (l * 64)
        sg = plsc.load_gather(seg_v, [row, col4]).astype(jnp.int32)
        m = plsc.load_gather(mask_v, [row, col4])
        e = err_v[r, pl.ds(l * 16, 16)]
        pos = jnp.where((m > 0.0) & (m < 0.5), 1.0, 0.0)
        base = sg * 16 + lane
        plsc.addupdate_scatter(table, [base], ones_f)
        plsc.addupdate_scatter(table, [base + (_NSEG * 16)], e)
        plsc.addupdate_scatter(table, [base + (2 * _NSEG * 16)], pos)

    @plsc.parallel_loop(0, _RROWS, unroll=2)
    def _loop(r):
        for l in range(_WE // 16):
            step(r, l)

    pltpu.sync_copy(table, out_hbm.at[wid])


def _sc_segsum(segr, maskr, err2d):
    mesh = plsc.VectorSubcoreMesh(core_axis_name="c", subcore_axis_name="s")
    fn = functools.partial(
        pl.kernel,
        mesh=mesh,
        compiler_params=pltpu.CompilerParams(needs_layout_passes=False),
        out_type=jax.ShapeDtypeStruct((_NW, 3 * _NSEG * 16), jnp.float32),
        scratch_types=[
            pltpu.VMEM((_RROWS, _W), jnp.float32),
            pltpu.VMEM((_RROWS, _W), jnp.float32),
            pltpu.VMEM((_RROWS, _WE), jnp.float32),
            pltpu.VMEM((3 * _NSEG * 16,), jnp.float32),
            pltpu.SemaphoreType.DMA,
        ],
    )(_sc_body)
    return fn(segr, maskr, err2d)


def _epi_body(p_ref, s_ref, o_ref):
    p = p_ref[...]
    r = lax.broadcasted_iota(jnp.int32, (3 * _NSEG * 16, 3 * _NSEG), 0)
    c = lax.broadcasted_iota(jnp.int32, (3 * _NSEG * 16, 3 * _NSEG), 1)
    gm = (r // 16 == c).astype(jnp.float32)
    t = jax.lax.dot(p, gm, precision=jax.lax.Precision.HIGHEST)
    num = 0.0
    den = 0.0
    for b in range(_B):
        g = t[4 * b] + t[4 * b + 1] + t[4 * b + 2] + t[4 * b + 3]
        counts = g[0:_NSEG]
        errs = g[_NSEG:2 * _NSEG]
        pos = g[2 * _NSEG:3 * _NSEG]
        cm = jnp.maximum(counts, 1.0)
        mean_err = errs / cm
        valid = (counts / _NPIX) >= 0.01
        is_pos = (pos / cm) > 0.01
        sel = jnp.where(valid & is_pos, 1.0, 0.0)
        num += jnp.sum(mean_err * sel)
        den += jnp.sum(sel)
    o_ref[0] = s_ref[0] / jnp.maximum(s_ref[1], 1.0) + num / jnp.maximum(den, 1.0)


def _epilogue(partials, sums):
    return pl.pallas_call(
        _epi_body,
        in_specs=[
            pl.BlockSpec(memory_space=pltpu.VMEM),
            pl.BlockSpec(memory_space=pltpu.SMEM),
        ],
        out_specs=pl.BlockSpec(memory_space=pltpu.SMEM),
        out_shape=jax.ShapeDtypeStruct((1,), jnp.float32),
    )(partials, sums)


def kernel(outputs, inputs, enc1, dec1, masks, segs, confidence, iteration, epoch):
    err, seg_rows, mask_rows, sums = _dense_pass(
        outputs, inputs, masks, enc1, dec1, segs
    )
    nr = _B * _HE
    partials = _sc_segsum(
        seg_rows.reshape(nr, _W),
        mask_rows.reshape(nr, _W),
        err.reshape(nr, _WE),
    )
    loss = _epilogue(partials, sums)
    return loss[0]
